# Initial kernel scaffold; baseline (speedup 1.0000x reference)
import jax
import jax.numpy as jnp
from jax.experimental import pallas as pl


def kernel(x, edge_index, W1l, b1, W1r, W2l, b2, W2r, Ws, Wy):
    raise NotImplementedError("write your pallas kernel here")


# speedup vs baseline: 2.9269x; 2.9269x over previous
import functools

import jax
import jax.numpy as jnp
from jax import lax
from jax.experimental import pallas as pl
from jax.experimental.pallas import tpu as pltpu
from jax.experimental.pallas import tpu_sc as plsc

NN = 10000
EE = 320000
FIN = 128
FH = 256
NC = 2
NS = 16
CH = 128
K1 = 80
K2 = 2 * K1
KP = 4
EPAD = NC * NS * K1 * CH
NPAD = 10112
RPT = NPAD // NS


@functools.cache
def _mesh():
    return plsc.VectorSubcoreMesh(core_axis_name="c", subcore_axis_name="s",
                                  num_cores=NC, num_subcores=NS)


def _zero_rows_buf(rows):
    def z(t, _):
        rows[t // 8, pl.ds((t % 8) * 16, 16)] = jnp.zeros((16,), jnp.float32)
        return 0
    lax.fori_loop(0, CH * 8, z, 0)


def _chunks():
    out = []
    o = 0
    while o < RPT:
        out.append((o, min(CH, RPT - o)))
        o += CH
    return out


def _zero_acc_slice(rows, acc_sh, tbase):
    for o, c in _chunks():
        pltpu.sync_copy(rows.at[pl.ds(0, c)], acc_sh.at[pl.ds(tbase + o, c)])


def _dump_acc_slice(rows, acc_sh, tbase, out_hbm):
    for o, c in _chunks():
        pltpu.sync_copy(acc_sh.at[pl.ds(tbase + o, c)], rows.at[pl.ds(0, c)])
        pltpu.sync_copy(rows.at[pl.ds(0, c)], out_hbm.at[pl.ds(tbase + o, c)])


def _sc_agg1_body(x_hbm, srcp_hbm, dstp_hbm, p_out, deg_out,
                  sidx, didx, rows, degtile, acc_sh, sem):
    cid = lax.axis_index("c")
    sid = lax.axis_index("s")
    wid = sid * NC + cid
    tbase = sid * RPT

    _zero_rows_buf(rows)

    def zdeg(t, _):
        degtile[pl.ds(t * 16, 16)] = jnp.zeros((16,), jnp.float32)
        return 0
    lax.fori_loop(0, NPAD // 16, zdeg, 0)

    _zero_acc_slice(rows, acc_sh, tbase)
    plsc.subcore_barrier()

    ones16 = jnp.ones((16,), jnp.float32)

    for hp in range(K1 // KP):
        pltpu.sync_copy(srcp_hbm.at[pl.ds(wid * K1 + hp * KP, KP)], sidx)
        pltpu.sync_copy(dstp_hbm.at[pl.ds(wid * K1 + hp * KP, KP)], didx)

        def step(g, _):
            pltpu.async_copy(x_hbm.at[sidx.at[g]], rows, sem).wait()
            pltpu.sync_copy(rows, acc_sh.at[didx.at[g]], add=True)

            def dstep(j, _):
                d = didx[g, pl.ds(j * 16, 16)]
                plsc.addupdate_scatter(degtile, [d], ones16)
                return 0
            lax.fori_loop(0, CH // 16, dstep, 0)
            return 0
        lax.fori_loop(0, KP, step, 0)

    plsc.subcore_barrier()
    _dump_acc_slice(rows, acc_sh, tbase, p_out.at[cid])
    pltpu.sync_copy(degtile, deg_out.at[cid, sid])


@functools.cache
def _sc_agg1():
  return pl.kernel(
    _sc_agg1_body,
    out_type=[jax.ShapeDtypeStruct((NC, NPAD, FIN), jnp.float32),
              jax.ShapeDtypeStruct((NC, NS, NPAD), jnp.float32)],
    mesh=_mesh(),
    compiler_params=pltpu.CompilerParams(needs_layout_passes=False),
    scratch_types=[
        pltpu.VMEM((KP, CH), jnp.int32),
        pltpu.VMEM((KP, CH), jnp.int32),
        pltpu.VMEM((CH, FIN), jnp.float32),
        pltpu.VMEM((NPAD,), jnp.float32),
        pltpu.VMEM_SHARED((NPAD, FIN), jnp.float32),
        pltpu.SemaphoreType.DMA,
    ],
  )


def _sc_agg2_body(h2v_hbm, srcp_hbm, dstp_hbm, a_out,
                  sidx, didx, scidx, rows, acc_sh, sem):
    cid = lax.axis_index("c")
    sid = lax.axis_index("s")
    tbase = sid * RPT

    _zero_rows_buf(rows)
    _zero_acc_slice(rows, acc_sh, tbase)
    plsc.subcore_barrier()

    for hp in range(K2 // KP):
        pltpu.sync_copy(srcp_hbm.at[pl.ds(sid * K2 + hp * KP, KP)], sidx)
        pltpu.sync_copy(dstp_hbm.at[pl.ds(sid * K2 + hp * KP, KP)], didx)

        def step(g, _):
            def scale(j, _):
                v = sidx[g, pl.ds(j * 16, 16)]
                scidx[pl.ds(j * 16, 16)] = v * 2 + cid
                return 0
            lax.fori_loop(0, CH // 16, scale, 0)
            pltpu.async_copy(h2v_hbm.at[scidx], rows, sem).wait()
            pltpu.sync_copy(rows, acc_sh.at[didx.at[g]], add=True)
            return 0
        lax.fori_loop(0, KP, step, 0)

    plsc.subcore_barrier()
    _dump_acc_slice(rows, acc_sh, tbase, a_out.at[cid])


@functools.cache
def _sc_agg2():
  return pl.kernel(
    _sc_agg2_body,
    out_type=jax.ShapeDtypeStruct((NC, NPAD, FIN), jnp.float32),
    mesh=_mesh(),
    compiler_params=pltpu.CompilerParams(needs_layout_passes=False),
    scratch_types=[
        pltpu.VMEM((KP, CH), jnp.int32),
        pltpu.VMEM((KP, CH), jnp.int32),
        pltpu.VMEM((CH,), jnp.int32),
        pltpu.VMEM((CH, FIN), jnp.float32),
        pltpu.VMEM_SHARED((NPAD, FIN), jnp.float32),
        pltpu.SemaphoreType.DMA,
    ],
  )


BN = 2048


def _rdeg(d_ref):
    deg = jnp.sum(d_ref[...], axis=(0, 1))[:, None]
    return 1.0 / jnp.maximum(deg, 1.0)


def _tc1_body(x_ref, p0_ref, p1_ref, d_ref, wl_ref, wr_ref, b_ref, h_ref):
    agg = (p0_ref[0] + p1_ref[0]) * _rdeg(d_ref)
    h = jnp.dot(agg, wl_ref[...], preferred_element_type=jnp.float32)
    h = h + jnp.dot(x_ref[...], wr_ref[...], preferred_element_type=jnp.float32)
    h = h + b_ref[...]
    h_ref[...] = jnp.maximum(h, 0.0)


_tc1 = pl.pallas_call(
    _tc1_body,
    grid=(pl.cdiv(NN, BN),),
    in_specs=[
        pl.BlockSpec((BN, FIN), lambda i: (i, 0)),
        pl.BlockSpec((1, BN, FIN), lambda i: (0, i, 0)),
        pl.BlockSpec((1, BN, FIN), lambda i: (1, i, 0)),
        pl.BlockSpec((NC, NS, BN), lambda i: (0, 0, i)),
        pl.BlockSpec((FIN, FH), lambda i: (0, 0)),
        pl.BlockSpec((FIN, FH), lambda i: (0, 0)),
        pl.BlockSpec((1, FH), lambda i: (0, 0)),
    ],
    out_specs=pl.BlockSpec((BN, FH), lambda i: (i, 0)),
    out_shape=jax.ShapeDtypeStruct((NN, FH), jnp.float32),
)


def _tc2_body(h_ref, a0_ref, a1_ref, d_ref, wl0_ref, wl1_ref,
              wr_ref, b_ref, wsy_ref, h2_ref, osy_ref):
    rdeg = _rdeg(d_ref)
    h2 = jnp.dot(a0_ref[0] * rdeg, wl0_ref[...],
                 preferred_element_type=jnp.float32)
    h2 = h2 + jnp.dot(a1_ref[0] * rdeg, wl1_ref[...],
                      preferred_element_type=jnp.float32)
    h2 = h2 + jnp.dot(h_ref[...], wr_ref[...],
                      preferred_element_type=jnp.float32)
    h2 = h2 + b_ref[...]
    h2_ref[...] = h2
    osy_ref[...] = jnp.dot(h2, wsy_ref[...], preferred_element_type=jnp.float32)


_tc2 = pl.pallas_call(
    _tc2_body,
    grid=(pl.cdiv(NN, BN),),
    in_specs=[
        pl.BlockSpec((BN, FH), lambda i: (i, 0)),
        pl.BlockSpec((1, BN, FIN), lambda i: (0, i, 0)),
        pl.BlockSpec((1, BN, FIN), lambda i: (1, i, 0)),
        pl.BlockSpec((NC, NS, BN), lambda i: (0, 0, i)),
        pl.BlockSpec((FIN, FH), lambda i: (0, 0)),
        pl.BlockSpec((FIN, FH), lambda i: (0, 0)),
        pl.BlockSpec((FH, FH), lambda i: (0, 0)),
        pl.BlockSpec((1, FH), lambda i: (0, 0)),
        pl.BlockSpec((FH, FIN), lambda i: (0, 0)),
    ],
    out_specs=[
        pl.BlockSpec((BN, FH), lambda i: (i, 0)),
        pl.BlockSpec((BN, FIN), lambda i: (i, 0)),
    ],
    out_shape=[
        jax.ShapeDtypeStruct((NN, FH), jnp.float32),
        jax.ShapeDtypeStruct((NN, FIN), jnp.float32),
    ],
)


def kernel(x, edge_index, W1l, b1, W1r, W2l, b2, W2r, Ws, Wy):
    src = edge_index[0]
    dst = edge_index[1]
    pad = EPAD - EE
    srcp = jnp.concatenate([src, jnp.zeros((pad,), jnp.int32)]
                           ).reshape(EPAD // CH, CH)
    dstp = jnp.concatenate([dst, jnp.full((pad,), NN, jnp.int32)]
                           ).reshape(EPAD // CH, CH)

    p, degp = _sc_agg1()(x, srcp, dstp)
    h = _tc1(x, p, p, degp, W1l.T, W1r.T, b1.reshape(1, FH))

    a2 = _sc_agg2()(h.reshape(2 * NN, FIN), srcp, dstp)

    wsy = jnp.concatenate([Ws, Wy], axis=0)
    wsy_pad = jnp.pad(wsy, ((0, FIN - wsy.shape[0]), (0, 0))).T
    w2lt = W2l.T
    h2, osy = _tc2(h, a2, a2, degp, w2lt[:FIN], w2lt[FIN:], W2r.T,
                   b2.reshape(1, FH), wsy_pad)
    return osy[:, :2], osy[:, 2:42], h2


# speedup vs baseline: 3.3948x; 1.1599x over previous
import functools

import jax
import jax.numpy as jnp
from jax import lax
from jax.experimental import pallas as pl
from jax.experimental.pallas import tpu as pltpu
from jax.experimental.pallas import tpu_sc as plsc

NN = 10000
EE = 320000
FIN = 128
FH = 256
NC = 2
NS = 16
CH = 64
PC = 16
K1 = 160
K2 = 2 * K1
EPAD = NC * NS * K1 * CH
NPAD = 10112
RPT = NPAD // NS


@functools.cache
def _mesh():
    return plsc.VectorSubcoreMesh(core_axis_name="c", subcore_axis_name="s",
                                  num_cores=NC, num_subcores=NS)


def _zero_buf(buf, nrows):
    def z(t, _):
        buf[t // 8, pl.ds((t % 8) * 16, 16)] = jnp.zeros((16,), jnp.float32)
        return 0
    lax.fori_loop(0, nrows * 8, z, 0)


def _chunks():
    out = []
    o = 0
    while o < RPT:
        out.append((o, min(CH, RPT - o)))
        o += CH
    return out


def _zero_acc_slice(buf, acc_sh, tbase):
    for o, c in _chunks():
        pltpu.sync_copy(buf.at[pl.ds(0, c)], acc_sh.at[pl.ds(tbase + o, c)])


def _dump_acc_slice(buf, acc_sh, tbase, out_hbm):
    for o, c in _chunks():
        pltpu.sync_copy(acc_sh.at[pl.ds(tbase + o, c)], buf.at[pl.ds(0, c)])
        pltpu.sync_copy(buf.at[pl.ds(0, c)], out_hbm.at[pl.ds(tbase + o, c)])


def _sc_agg1_body(x_hbm, srcp_hbm, dstp_hbm, p_out, deg_out,
                  sidx, didx, bufa, bufb, degtile, acc_sh,
                  ga, gb, sa, sb):
    cid = lax.axis_index("c")
    sid = lax.axis_index("s")
    wid = sid * NC + cid
    tbase = sid * RPT

    _zero_buf(bufa, CH)

    def zdeg(t, _):
        degtile[pl.ds(t * 16, 16)] = jnp.zeros((16,), jnp.float32)
        return 0
    lax.fori_loop(0, NPAD // 16, zdeg, 0)

    _zero_acc_slice(bufa, acc_sh, tbase)
    plsc.subcore_barrier()

    ones16 = jnp.ones((16,), jnp.float32)
    bufs = (bufa, bufb)
    gsem = (ga, gb)
    ssem = (sa, sb)

    def one_pass(hp, _):
        base = wid * K1 + hp * PC
        pltpu.sync_copy(srcp_hbm.at[pl.ds(base, PC)], sidx)
        pltpu.sync_copy(dstp_hbm.at[pl.ds(base, PC)], didx)
        g0 = pltpu.async_copy(x_hbm.at[sidx.at[0]], bufa, ga)
        g1 = pltpu.async_copy(x_hbm.at[sidx.at[1]], bufb, gb)
        gd = [g0, g1]
        sd = [None, None]
        for c in range(PC):
            b = c % 2
            gd[b].wait()
            sd[b] = pltpu.async_copy(bufs[b], acc_sh.at[didx.at[c]],
                                     ssem[b], add=True)
            for j in range(CH // 16):
                d = didx[c, pl.ds(j * 16, 16)]
                plsc.addupdate_scatter(degtile, [d], ones16)
            if c + 2 < PC:
                sd[b].wait()
                gd[b] = pltpu.async_copy(x_hbm.at[sidx.at[c + 2]],
                                         bufs[b], gsem[b])
        sd[0].wait()
        sd[1].wait()
        return 0
    lax.fori_loop(0, K1 // PC, one_pass, 0)

    plsc.subcore_barrier()
    _dump_acc_slice(bufa, acc_sh, tbase, p_out.at[cid])
    pltpu.sync_copy(degtile, deg_out.at[cid, sid])


@functools.cache
def _sc_agg1():
  return pl.kernel(
    _sc_agg1_body,
    out_type=[jax.ShapeDtypeStruct((NC, NPAD, FIN), jnp.float32),
              jax.ShapeDtypeStruct((NC, NS, NPAD), jnp.float32)],
    mesh=_mesh(),
    compiler_params=pltpu.CompilerParams(needs_layout_passes=False),
    scratch_types=[
        pltpu.VMEM((PC, CH), jnp.int32),
        pltpu.VMEM((PC, CH), jnp.int32),
        pltpu.VMEM((CH, FIN), jnp.float32),
        pltpu.VMEM((CH, FIN), jnp.float32),
        pltpu.VMEM((NPAD,), jnp.float32),
        pltpu.VMEM_SHARED((NPAD, FIN), jnp.float32),
        pltpu.SemaphoreType.DMA,
        pltpu.SemaphoreType.DMA,
        pltpu.SemaphoreType.DMA,
        pltpu.SemaphoreType.DMA,
    ],
  )


def _sc_agg2_body(h2v_hbm, srcp_hbm, dstp_hbm, a_out,
                  sidx, didx, sca, scb, bufa, bufb, acc_sh,
                  ga, gb, sa, sb):
    cid = lax.axis_index("c")
    sid = lax.axis_index("s")
    tbase = sid * RPT

    _zero_buf(bufa, CH)
    _zero_acc_slice(bufa, acc_sh, tbase)
    plsc.subcore_barrier()

    bufs = (bufa, bufb)
    scx = (sca, scb)
    gsem = (ga, gb)
    ssem = (sa, sb)

    def scale_into(dst, c):
        def scale(j, _):
            v = sidx[c, pl.ds(j * 16, 16)]
            dst[pl.ds(j * 16, 16)] = v * 2 + cid
            return 0
        lax.fori_loop(0, CH // 16, scale, 0)

    def one_pass(hp, _):
        base = sid * K2 + hp * PC
        pltpu.sync_copy(srcp_hbm.at[pl.ds(base, PC)], sidx)
        pltpu.sync_copy(dstp_hbm.at[pl.ds(base, PC)], didx)
        scale_into(sca, 0)
        scale_into(scb, 1)
        g0 = pltpu.async_copy(h2v_hbm.at[sca], bufa, ga)
        g1 = pltpu.async_copy(h2v_hbm.at[scb], bufb, gb)
        gd = [g0, g1]
        sd = [None, None]
        for c in range(PC):
            b = c % 2
            gd[b].wait()
            sd[b] = pltpu.async_copy(bufs[b], acc_sh.at[didx.at[c]],
                                     ssem[b], add=True)
            if c + 2 < PC:
                sd[b].wait()
                scale_into(scx[b], c + 2)
                gd[b] = pltpu.async_copy(h2v_hbm.at[scx[b]],
                                         bufs[b], gsem[b])
        sd[0].wait()
        sd[1].wait()
        return 0
    lax.fori_loop(0, K2 // PC, one_pass, 0)

    plsc.subcore_barrier()
    _dump_acc_slice(bufa, acc_sh, tbase, a_out.at[cid])


@functools.cache
def _sc_agg2():
  return pl.kernel(
    _sc_agg2_body,
    out_type=jax.ShapeDtypeStruct((NC, NPAD, FIN), jnp.float32),
    mesh=_mesh(),
    compiler_params=pltpu.CompilerParams(needs_layout_passes=False),
    scratch_types=[
        pltpu.VMEM((PC, CH), jnp.int32),
        pltpu.VMEM((PC, CH), jnp.int32),
        pltpu.VMEM((CH,), jnp.int32),
        pltpu.VMEM((CH,), jnp.int32),
        pltpu.VMEM((CH, FIN), jnp.float32),
        pltpu.VMEM((CH, FIN), jnp.float32),
        pltpu.VMEM_SHARED((NPAD, FIN), jnp.float32),
        pltpu.SemaphoreType.DMA,
        pltpu.SemaphoreType.DMA,
        pltpu.SemaphoreType.DMA,
        pltpu.SemaphoreType.DMA,
    ],
  )


BN = 2048


def _rdeg(d_ref):
    deg = jnp.sum(d_ref[...], axis=(0, 1))[:, None]
    return 1.0 / jnp.maximum(deg, 1.0)


def _tc1_body(x_ref, p0_ref, p1_ref, d_ref, wl_ref, wr_ref, b_ref, h_ref):
    agg = (p0_ref[0] + p1_ref[0]) * _rdeg(d_ref)
    h = jnp.dot(agg, wl_ref[...], preferred_element_type=jnp.float32)
    h = h + jnp.dot(x_ref[...], wr_ref[...], preferred_element_type=jnp.float32)
    h = h + b_ref[...]
    h_ref[...] = jnp.maximum(h, 0.0)


_tc1 = pl.pallas_call(
    _tc1_body,
    grid=(pl.cdiv(NN, BN),),
    in_specs=[
        pl.BlockSpec((BN, FIN), lambda i: (i, 0)),
        pl.BlockSpec((1, BN, FIN), lambda i: (0, i, 0)),
        pl.BlockSpec((1, BN, FIN), lambda i: (1, i, 0)),
        pl.BlockSpec((NC, NS, BN), lambda i: (0, 0, i)),
        pl.BlockSpec((FIN, FH), lambda i: (0, 0)),
        pl.BlockSpec((FIN, FH), lambda i: (0, 0)),
        pl.BlockSpec((1, FH), lambda i: (0, 0)),
    ],
    out_specs=pl.BlockSpec((BN, FH), lambda i: (i, 0)),
    out_shape=jax.ShapeDtypeStruct((NN, FH), jnp.float32),
)


def _tc2_body(h_ref, a0_ref, a1_ref, d_ref, wl0_ref, wl1_ref,
              wr_ref, b_ref, wsy_ref, h2_ref, osy_ref):
    rdeg = _rdeg(d_ref)
    h2 = jnp.dot(a0_ref[0] * rdeg, wl0_ref[...],
                 preferred_element_type=jnp.float32)
    h2 = h2 + jnp.dot(a1_ref[0] * rdeg, wl1_ref[...],
                      preferred_element_type=jnp.float32)
    h2 = h2 + jnp.dot(h_ref[...], wr_ref[...],
                      preferred_element_type=jnp.float32)
    h2 = h2 + b_ref[...]
    h2_ref[...] = h2
    osy_ref[...] = jnp.dot(h2, wsy_ref[...], preferred_element_type=jnp.float32)


_tc2 = pl.pallas_call(
    _tc2_body,
    grid=(pl.cdiv(NN, BN),),
    in_specs=[
        pl.BlockSpec((BN, FH), lambda i: (i, 0)),
        pl.BlockSpec((1, BN, FIN), lambda i: (0, i, 0)),
        pl.BlockSpec((1, BN, FIN), lambda i: (1, i, 0)),
        pl.BlockSpec((NC, NS, BN), lambda i: (0, 0, i)),
        pl.BlockSpec((FIN, FH), lambda i: (0, 0)),
        pl.BlockSpec((FIN, FH), lambda i: (0, 0)),
        pl.BlockSpec((FH, FH), lambda i: (0, 0)),
        pl.BlockSpec((1, FH), lambda i: (0, 0)),
        pl.BlockSpec((FH, FIN), lambda i: (0, 0)),
    ],
    out_specs=[
        pl.BlockSpec((BN, FH), lambda i: (i, 0)),
        pl.BlockSpec((BN, FIN), lambda i: (i, 0)),
    ],
    out_shape=[
        jax.ShapeDtypeStruct((NN, FH), jnp.float32),
        jax.ShapeDtypeStruct((NN, FIN), jnp.float32),
    ],
)


def kernel(x, edge_index, W1l, b1, W1r, W2l, b2, W2r, Ws, Wy):
    src = edge_index[0]
    dst = edge_index[1]
    pad = EPAD - EE
    srcp = jnp.concatenate([src, jnp.zeros((pad,), jnp.int32)]
                           ).reshape(EPAD // CH, CH)
    dstp = jnp.concatenate([dst, jnp.full((pad,), NN, jnp.int32)]
                           ).reshape(EPAD // CH, CH)

    p, degp = _sc_agg1()(x, srcp, dstp)
    h = _tc1(x, p, p, degp, W1l.T, W1r.T, b1.reshape(1, FH))

    a2 = _sc_agg2()(h.reshape(2 * NN, FIN), srcp, dstp)

    wsy = jnp.concatenate([Ws, Wy], axis=0)
    wsy_pad = jnp.pad(wsy, ((0, FIN - wsy.shape[0]), (0, 0))).T
    w2lt = W2l.T
    h2, osy = _tc2(h, a2, a2, degp, w2lt[:FIN], w2lt[FIN:], W2r.T,
                   b2.reshape(1, FH), wsy_pad)
    return osy[:, :2], osy[:, 2:42], h2
